# baseline (speedup 1.0000x reference)
import jax
import jax.numpy as jnp
from jax import lax
from jax.experimental import pallas as pl
from jax.experimental.pallas import tpu as pltpu
from jax.experimental.pallas import tpu_sc as plsc

BATCH = 16384
DIM = 64
LANES = 16
NUM_WORKERS = 32
BPW = BATCH // NUM_WORKERS
CHUNK = 128
NCHUNK = BPW // CHUNK
PAIRS = 500000


def _body(s_hbm, r_hbm, d_hbm, emb_hbm, rel_hbm, out_hbm,
          sidx, ridx, didx, ssub, rsub, dsub,
          hbuf, rbuf, tbuf, outv, sem):
    cid = lax.axis_index("c")
    sid = lax.axis_index("s")
    wid = sid * 2 + cid
    base = wid * BPW

    lanes = lax.iota(jnp.int32, LANES)

    pltpu.sync_copy(s_hbm.at[pl.ds(base, BPW)], sidx)
    pltpu.sync_copy(r_hbm.at[pl.ds(base, BPW)], ridx)
    pltpu.sync_copy(d_hbm.at[pl.ds(base, BPW)], didx)

    def split(i, carry):
        sl = pl.ds(i * LANES, LANES)
        for idx, sub in ((sidx, ssub), (ridx, rsub), (didx, dsub)):
            v = idx[sl]
            sub[sl] = lax.shift_left(lax.bitwise_and(v, 1), 6)
            idx[sl] = lax.shift_right_logical(v, 1)
        return carry

    lax.fori_loop(0, BPW // LANES, split, 0)

    def fire(c, b):
        sl = pl.ds(c * CHUNK, CHUNK)
        pltpu.async_copy(emb_hbm.at[sidx.at[sl]], hbuf.at[b], sem)
        pltpu.async_copy(rel_hbm.at[ridx.at[sl]], rbuf.at[b], sem)
        pltpu.async_copy(emb_hbm.at[didx.at[sl]], tbuf.at[b], sem)

    def drain(b):
        pltpu.make_async_copy(emb_hbm.at[pl.ds(0, CHUNK)], hbuf.at[b], sem).wait()
        pltpu.make_async_copy(emb_hbm.at[pl.ds(0, CHUNK)], rbuf.at[b], sem).wait()
        pltpu.make_async_copy(emb_hbm.at[pl.ds(0, CHUNK)], tbuf.at[b], sem).wait()

    def compute(c, b):
        def grp(g, carry):
            gsl = pl.ds(c * CHUNK + g * LANES, LANES)
            tri = g * LANES + lanes
            hs = ssub[gsl]
            rs = rsub[gsl]
            ds_ = dsub[gsl]

            def col(j, acc):
                cv = jnp.zeros((LANES,), jnp.int32) + j
                h = plsc.load_gather(hbuf.at[b], [tri, hs + cv])
                rr = plsc.load_gather(rbuf.at[b], [tri, rs + cv])
                t = plsc.load_gather(tbuf.at[b], [tri, ds_ + cv])
                return acc + jnp.abs(h + rr - t)

            acc = lax.fori_loop(0, DIM, col, jnp.zeros((LANES,), jnp.float32))
            outv[gsl] = 1.0 / (1.0 + jnp.exp(acc))
            return carry

        lax.fori_loop(0, CHUNK // LANES, grp, 0)

    fire(0, 0)

    def chunk_loop(c, carry):
        b = lax.rem(c, 2)
        drain(b)

        @pl.when(c < NCHUNK - 1)
        def _():
            fire(c + 1, 1 - b)

        compute(c, b)
        return carry

    lax.fori_loop(0, NCHUNK, chunk_loop, 0)

    pltpu.sync_copy(outv, out_hbm.at[pl.ds(base, BPW)])


def kernel(x, emb, rel_emb):
    mesh = plsc.VectorSubcoreMesh(core_axis_name="c", subcore_axis_name="s")
    run = pl.kernel(
        _body,
        out_type=jax.ShapeDtypeStruct((BATCH,), jnp.float32),
        mesh=mesh,
        compiler_params=pltpu.CompilerParams(
            needs_layout_passes=False, use_tc_tiling_on_sc=False),
        scratch_types=[
            pltpu.VMEM((BPW,), jnp.int32),
            pltpu.VMEM((BPW,), jnp.int32),
            pltpu.VMEM((BPW,), jnp.int32),
            pltpu.VMEM((BPW,), jnp.int32),
            pltpu.VMEM((BPW,), jnp.int32),
            pltpu.VMEM((BPW,), jnp.int32),
            pltpu.VMEM((2, CHUNK, 2 * DIM), jnp.float32),
            pltpu.VMEM((2, CHUNK, 2 * DIM), jnp.float32),
            pltpu.VMEM((2, CHUNK, 2 * DIM), jnp.float32),
            pltpu.VMEM((BPW,), jnp.float32),
            pltpu.SemaphoreType.DMA,
        ],
    )
    xi = x.astype(jnp.int32)
    emb2 = emb.reshape(PAIRS, 2 * DIM)
    rel2 = rel_emb.reshape(PAIRS, 2 * DIM)
    return run(xi[:, 0], xi[:, 1], xi[:, 2], emb2, rel2)


# speedup vs baseline: 1.0007x; 1.0007x over previous
import jax
import jax.numpy as jnp
from jax import lax
from jax.experimental import pallas as pl
from jax.experimental.pallas import tpu as pltpu
from jax.experimental.pallas import tpu_sc as plsc

BATCH = 16384
DIM = 64
LANES = 16
NUM_WORKERS = 32
BPW = BATCH // NUM_WORKERS
CHUNK = 128
NCHUNK = BPW // CHUNK
PAIRS = 500000


def _body(s_hbm, r_hbm, d_hbm, emb_hbm, rel_hbm, out_hbm,
          sidx, ridx, didx, ssub, rsub, dsub,
          hbuf, rbuf, tbuf, outv, sem):
    cid = lax.axis_index("c")
    sid = lax.axis_index("s")
    wid = sid * 2 + cid
    base = wid * BPW

    lanes = lax.iota(jnp.int32, LANES)

    pltpu.sync_copy(s_hbm.at[pl.ds(base, BPW)], sidx)
    pltpu.sync_copy(r_hbm.at[pl.ds(base, BPW)], ridx)
    pltpu.sync_copy(d_hbm.at[pl.ds(base, BPW)], didx)

    def split(i, carry):
        sl = pl.ds(i * LANES, LANES)
        for idx, sub in ((sidx, ssub), (ridx, rsub), (didx, dsub)):
            v = idx[sl]
            sub[sl] = lax.shift_left(lax.bitwise_and(v, 1), 6)
            idx[sl] = lax.shift_right_logical(v, 1)
        return carry

    lax.fori_loop(0, BPW // LANES, split, 0)

    def fire(c, b):
        sl = pl.ds(c * CHUNK, CHUNK)
        pltpu.async_copy(emb_hbm.at[sidx.at[sl]], hbuf.at[b], sem)
        pltpu.async_copy(rel_hbm.at[ridx.at[sl]], rbuf.at[b], sem)
        pltpu.async_copy(emb_hbm.at[didx.at[sl]], tbuf.at[b], sem)

    def drain(b):
        pltpu.make_async_copy(emb_hbm.at[pl.ds(0, CHUNK)], hbuf.at[b], sem).wait()
        pltpu.make_async_copy(emb_hbm.at[pl.ds(0, CHUNK)], rbuf.at[b], sem).wait()
        pltpu.make_async_copy(emb_hbm.at[pl.ds(0, CHUNK)], tbuf.at[b], sem).wait()

    def compute(c, b):
        def grp(g, carry):
            gsl = pl.ds(c * CHUNK + g * LANES, LANES)
            tri = g * LANES + lanes
            hs = ssub[gsl]
            rs = rsub[gsl]
            ds_ = dsub[gsl]

            def col(j, acc):
                cv = jnp.zeros((LANES,), jnp.int32) + j
                h = plsc.load_gather(hbuf.at[b], [tri, hs + cv])
                rr = plsc.load_gather(rbuf.at[b], [tri, rs + cv])
                t = plsc.load_gather(tbuf.at[b], [tri, ds_ + cv])
                return acc + jnp.abs(h + rr - t)

            acc = lax.fori_loop(0, DIM, col, jnp.zeros((LANES,), jnp.float32))
            outv[gsl] = 1.0 / (1.0 + jnp.exp(acc))
            return carry

        lax.fori_loop(0, CHUNK // LANES, grp, 0)

    fire(0, 0)

    def chunk_loop(c, carry):
        b = lax.rem(c, 2)
        drain(b)

        @pl.when(c < NCHUNK - 1)
        def _():
            fire(c + 1, 1 - b)

        compute(c, b)
        return carry

    lax.fori_loop(0, NCHUNK, chunk_loop, 0)

    pltpu.sync_copy(outv, out_hbm.at[pl.ds(base, BPW)])


def kernel(x, emb, rel_emb):
    mesh = plsc.VectorSubcoreMesh(core_axis_name="c", subcore_axis_name="s")
    run = pl.kernel(
        _body,
        out_type=jax.ShapeDtypeStruct((BATCH,), jnp.float32),
        mesh=mesh,
        compiler_params=pltpu.CompilerParams(
            needs_layout_passes=False, use_tc_tiling_on_sc=True),
        scratch_types=[
            pltpu.VMEM((BPW,), jnp.int32),
            pltpu.VMEM((BPW,), jnp.int32),
            pltpu.VMEM((BPW,), jnp.int32),
            pltpu.VMEM((BPW,), jnp.int32),
            pltpu.VMEM((BPW,), jnp.int32),
            pltpu.VMEM((BPW,), jnp.int32),
            pltpu.VMEM((2, CHUNK, 2 * DIM), jnp.float32),
            pltpu.VMEM((2, CHUNK, 2 * DIM), jnp.float32),
            pltpu.VMEM((2, CHUNK, 2 * DIM), jnp.float32),
            pltpu.VMEM((BPW,), jnp.float32),
            pltpu.SemaphoreType.DMA,
        ],
    )
    xi = x.astype(jnp.int32)
    emb2 = emb.reshape(PAIRS, 2 * DIM)
    rel2 = rel_emb.reshape(PAIRS, 2 * DIM)
    return run(xi[:, 0], xi[:, 1], xi[:, 2], emb2, rel2)


# speedup vs baseline: 1.6298x; 1.6286x over previous
import jax
import jax.numpy as jnp
from jax import lax
from jax.experimental import pallas as pl
from jax.experimental.pallas import tpu as pltpu
from jax.experimental.pallas import tpu_sc as plsc

BATCH = 16384
DIM = 64
LANES = 16
NUM_WORKERS = 32
BPW = BATCH // NUM_WORKERS
G = 128
NST = BPW // G


def _body(s_hbm, r_hbm, d_hbm, emb_hbm, rel_hbm, out_hbm,
          sidx, ridx, didx, hbuf, rbuf, tbuf, outv, sem, sem2, sem3):
    cid = lax.axis_index("c")
    sid = lax.axis_index("s")
    wid = sid * 2 + cid
    base = wid * BPW

    lanes = lax.iota(jnp.int32, LANES)

    pltpu.sync_copy(s_hbm.at[pl.ds(base, BPW)], sidx)
    pltpu.sync_copy(r_hbm.at[pl.ds(base, BPW)], ridx)
    pltpu.sync_copy(d_hbm.at[pl.ds(base, BPW)], didx)

    def fire(stage, b):
        def fire_grp(g, carry):
            off = stage * G + g * LANES
            vs = sidx[pl.ds(off, LANES)]
            vr = ridx[pl.ds(off, LANES)]
            vd = didx[pl.ds(off, LANES)]
            for k in range(LANES):
                row = pl.ds((g * LANES + k) * DIM, DIM)
                pltpu.async_copy(emb_hbm.at[vs[k]], hbuf.at[b, row], sem)
                pltpu.async_copy(rel_hbm.at[vr[k]], rbuf.at[b, row], sem2)
                pltpu.async_copy(emb_hbm.at[vd[k]], tbuf.at[b, row], sem3)
            return carry

        lax.fori_loop(0, G // LANES, fire_grp, 0)

    def drain(b):
        pltpu.make_async_copy(out_hbm.at[pl.ds(0, G * DIM)], hbuf.at[b], sem).wait()
        pltpu.make_async_copy(out_hbm.at[pl.ds(0, G * DIM)], rbuf.at[b], sem2).wait()
        pltpu.make_async_copy(out_hbm.at[pl.ds(0, G * DIM)], tbuf.at[b], sem3).wait()


    def compute(stage, b):
        def cg(g, carry):
            acc = jnp.zeros((LANES,), jnp.float32)
            for k in range(LANES):
                row = g * LANES + k
                w = jnp.zeros((LANES,), jnp.float32)
                for j in range(DIM // LANES):
                    h = hbuf[b, pl.ds(row * DIM + j * LANES, LANES)]
                    rr = rbuf[b, pl.ds(row * DIM + j * LANES, LANES)]
                    t = tbuf[b, pl.ds(row * DIM + j * LANES, LANES)]
                    w = w + jnp.abs(h + rr - t)
                acc = jnp.where(lanes == k, jnp.sum(w), acc)
            outv[pl.ds(stage * G + g * LANES, LANES)] = 1.0 / (1.0 + jnp.exp(acc))
            return carry

        lax.fori_loop(0, G // LANES, cg, 0)

    fire(0, 0)

    def stage_loop(s, carry):
        b = lax.rem(s, 2)
        drain(b)

        @pl.when(s < NST - 1)
        def _():
            fire(s + 1, 1 - b)

        compute(s, b)
        return carry

    lax.fori_loop(0, NST, stage_loop, 0)

    pltpu.sync_copy(outv, out_hbm.at[pl.ds(base, BPW)])


def kernel(x, emb, rel_emb):
    mesh = plsc.VectorSubcoreMesh(core_axis_name="c", subcore_axis_name="s")
    run = pl.kernel(
        _body,
        out_type=jax.ShapeDtypeStruct((BATCH,), jnp.float32),
        mesh=mesh,
        compiler_params=pltpu.CompilerParams(needs_layout_passes=False),
        scratch_types=[
            pltpu.VMEM((BPW,), jnp.int32),
            pltpu.VMEM((BPW,), jnp.int32),
            pltpu.VMEM((BPW,), jnp.int32),
            pltpu.VMEM((2, G * DIM), jnp.float32),
            pltpu.VMEM((2, G * DIM), jnp.float32),
            pltpu.VMEM((2, G * DIM), jnp.float32),
            pltpu.VMEM((BPW,), jnp.float32),
            pltpu.SemaphoreType.DMA,
            pltpu.SemaphoreType.DMA,
            pltpu.SemaphoreType.DMA,
        ],
    )
    xi = x.astype(jnp.int32)
    return run(xi[:, 0], xi[:, 1], xi[:, 2], emb, rel_emb)
